# Initial kernel scaffold; baseline (speedup 1.0000x reference)
import jax
import jax.numpy as jnp
from jax.experimental import pallas as pl


def kernel(z_l, z_h, edge_index_l_h, We1, be1, We2, be2, Ww1, bw1, Ww2, bw2, Wn1, bn1, Wn2, bn2):
    raise NotImplementedError("write your pallas kernel here")


# speedup vs baseline: 1.4974x; 1.4974x over previous
import functools

import jax
import jax.numpy as jnp
from jax import lax
from jax.experimental import pallas as pl
from jax.experimental.pallas import tpu as pltpu
from jax.experimental.pallas import tpu_sc as plsc

FD = 13
MD = 16
HD = 32
PAD = 16

N_NODES = 100000
E_EDGES = 1600000

NC = 2
NS = 16
NW = NC * NS
PER_W = E_EDGES // NW
CHUNK = 2000

RPT = N_NODES // NS
ZB = 250
SCHUNK = 1000

EBLK = 8000
NBLK = 10000

_mesh = plsc.VectorSubcoreMesh(core_axis_name="c", subcore_axis_name="s")
_sc_params = pltpu.CompilerParams(use_tc_tiling_on_sc=False)


@functools.partial(
    pl.kernel,
    mesh=_mesh,
    out_type=[
        jax.ShapeDtypeStruct((E_EDGES, PAD), jnp.float32),
        jax.ShapeDtypeStruct((E_EDGES, PAD), jnp.float32),
    ],
    scratch_types=[
        pltpu.VMEM((CHUNK,), jnp.int32),
        pltpu.VMEM((CHUNK,), jnp.int32),
        pltpu.VMEM((CHUNK, PAD), jnp.float32),
        pltpu.VMEM((CHUNK, PAD), jnp.float32),
        pltpu.SemaphoreType.DMA,
    ],
    compiler_params=_sc_params,
)
def _gather_sc(zl_hbm, zh_hbm, src_hbm, tgt_hbm, outl_hbm, outh_hbm,
               idx_s, idx_t, rows_l, rows_h, sem):
    wid = lax.axis_index("s") * NC + lax.axis_index("c")
    base0 = wid * PER_W

    @pl.loop(0, PER_W, step=CHUNK)
    def _(off):
        base = base0 + off
        pltpu.sync_copy(src_hbm.at[pl.ds(base, CHUNK)], idx_s)
        pltpu.sync_copy(tgt_hbm.at[pl.ds(base, CHUNK)], idx_t)
        cl = pltpu.async_copy(zl_hbm.at[idx_s], rows_l, sem)
        ch = pltpu.async_copy(zh_hbm.at[idx_t], rows_h, sem)
        cl.wait()
        ch.wait()
        pltpu.sync_copy(rows_l, outl_hbm.at[pl.ds(base, CHUNK)])
        pltpu.sync_copy(rows_h, outh_hbm.at[pl.ds(base, CHUNK)])


@functools.partial(
    pl.kernel,
    mesh=_mesh,
    out_type=jax.ShapeDtypeStruct((NC, N_NODES, MD), jnp.float32),
    scratch_types=[
        pltpu.VMEM_SHARED((N_NODES, MD), jnp.float32),
        pltpu.VMEM((SCHUNK,), jnp.int32),
        pltpu.VMEM((SCHUNK, MD), jnp.float32),
        pltpu.VMEM((ZB, MD), jnp.float32),
    ],
    compiler_params=_sc_params,
)
def _scatter_sc(vals_hbm, tgt_hbm, out_hbm, acc, idx_v, vals_v, zbuf):
    cid = lax.axis_index("c")
    sid = lax.axis_index("s")

    @pl.loop(0, ZB)
    def _(i):
        zbuf.at[pl.ds(i, 1), pl.ds(0, MD)][...] = jnp.zeros((1, MD), jnp.float32)

    @pl.loop(0, RPT, step=ZB)
    def _(j):
        pltpu.sync_copy(zbuf, acc.at[pl.ds(sid * RPT + j, ZB)])

    plsc.subcore_barrier()

    base0 = (cid * NS + sid) * PER_W

    @pl.loop(0, PER_W, step=SCHUNK)
    def _(off):
        base = base0 + off
        pltpu.sync_copy(tgt_hbm.at[pl.ds(base, SCHUNK)], idx_v)
        pltpu.sync_copy(vals_hbm.at[pl.ds(base, SCHUNK)], vals_v)
        pltpu.sync_copy(vals_v, acc.at[idx_v], add=True)

    plsc.subcore_barrier()
    pltpu.sync_copy(acc.at[pl.ds(sid * RPT, RPT)],
                    out_hbm.at[cid, pl.ds(sid * RPT, RPT)])


def _edge_mlp_body(gl_ref, gh_ref, w1_ref, b1_ref, w2_ref, b2_ref, out_ref):
    gl = gl_ref[...]
    gh = gh_ref[...]
    d = gl[:, 0:3] - gh[:, 0:3]
    dist = jnp.sum(d * d, axis=1, keepdims=True)
    us = gl[:, 3:6]
    ut = gh[:, 3:6]
    c0 = us[:, 1:2] * ut[:, 2:3] - us[:, 2:3] * ut[:, 1:2]
    c1 = us[:, 2:3] * ut[:, 0:1] - us[:, 0:1] * ut[:, 2:3]
    c2 = us[:, 0:1] * ut[:, 1:2] - us[:, 1:2] * ut[:, 0:1]
    cr = jnp.concatenate([c0, c1, c2], axis=1)
    ac = jnp.sqrt(jnp.sum(cr * cr, axis=1, keepdims=True))
    pad8 = jnp.zeros((gl.shape[0], 8), jnp.float32)
    inp = jnp.concatenate([gl, gh, d, dist, cr, ac, pad8], axis=1)
    h = jnp.tanh(
        jnp.dot(inp, w1_ref[...], precision=lax.Precision.HIGHEST,
                preferred_element_type=jnp.float32) + b1_ref[...]
    )
    mw = jnp.dot(h, w2_ref[...], precision=lax.Precision.HIGHEST,
                 preferred_element_type=jnp.float32) + b2_ref[...]
    m = mw[:, 0:MD]
    wl = mw[:, MD:MD + 1]
    out_ref[...] = m * jax.nn.sigmoid(wl)


def _edge_mlp(gl, gh, w1, b1, w2, b2):
    grid = E_EDGES // EBLK
    return pl.pallas_call(
        _edge_mlp_body,
        grid=(grid,),
        in_specs=[
            pl.BlockSpec((EBLK, PAD), lambda i: (i, 0)),
            pl.BlockSpec((EBLK, PAD), lambda i: (i, 0)),
            pl.BlockSpec(w1.shape, lambda i: (0, 0)),
            pl.BlockSpec(b1.shape, lambda i: (0, 0)),
            pl.BlockSpec(w2.shape, lambda i: (0, 0)),
            pl.BlockSpec(b2.shape, lambda i: (0, 0)),
        ],
        out_specs=pl.BlockSpec((EBLK, MD), lambda i: (i, 0)),
        out_shape=jax.ShapeDtypeStruct((E_EDGES, MD), jnp.float32),
    )(gl, gh, w1, b1, w2, b2)


def _node_mlp_body(zh_ref, p_ref, wn1_ref, bn1_ref, wn2_ref, bn2_ref, out_ref):
    zh = zh_ref[...]
    p = p_ref[...]
    magg = p[0] + p[1]
    inp = jnp.concatenate([zh, magg], axis=1)
    h = jnp.tanh(
        jnp.dot(inp, wn1_ref[...], precision=lax.Precision.HIGHEST,
                preferred_element_type=jnp.float32) + bn1_ref[...]
    )
    out_ref[...] = jnp.dot(h, wn2_ref[...], precision=lax.Precision.HIGHEST,
                           preferred_element_type=jnp.float32) + bn2_ref[...]


def _node_mlp(zh, partials, wn1, bn1, wn2, bn2):
    grid = N_NODES // NBLK
    return pl.pallas_call(
        _node_mlp_body,
        grid=(grid,),
        in_specs=[
            pl.BlockSpec((NBLK, FD), lambda i: (i, 0)),
            pl.BlockSpec((NC, NBLK, MD), lambda i: (0, i, 0)),
            pl.BlockSpec(wn1.shape, lambda i: (0, 0)),
            pl.BlockSpec(bn1.shape, lambda i: (0, 0)),
            pl.BlockSpec(wn2.shape, lambda i: (0, 0)),
            pl.BlockSpec(bn2.shape, lambda i: (0, 0)),
        ],
        out_specs=pl.BlockSpec((NBLK, FD), lambda i: (i, 0)),
        out_shape=jax.ShapeDtypeStruct((N_NODES, FD), jnp.float32),
    )(zh, partials, wn1, bn1, wn2, bn2)


def kernel(z_l, z_h, edge_index_l_h, We1, be1, We2, be2,
           Ww1, bw1, Ww2, bw2, Wn1, bn1, Wn2, bn2):
    zl = z_l[0]
    zh = z_h[0]
    src = edge_index_l_h[0, 0].astype(jnp.int32)
    tgt = edge_index_l_h[0, 1].astype(jnp.int32)

    zl_pad = jnp.pad(zl, ((0, 0), (0, PAD - FD)))
    zh_pad = jnp.pad(zh, ((0, 0), (0, PAD - FD)))

    w1 = jnp.concatenate([We1, Ww1], axis=1)
    b1 = jnp.concatenate([be1, bw1])[None, :]
    w1c = jnp.zeros((48, 2 * HD), jnp.float32)
    w1c = w1c.at[0:FD].set(w1[0:FD])
    w1c = w1c.at[PAD:PAD + FD].set(w1[FD:2 * FD])
    w1c = w1c.at[2 * PAD:2 * PAD + 8].set(w1[2 * FD:2 * FD + 8])
    w2c = jnp.zeros((2 * HD, MD + 1), jnp.float32)
    w2c = w2c.at[0:HD, 0:MD].set(We2)
    w2c = w2c.at[HD:2 * HD, MD].set(Ww2[:, 0])
    b2c = jnp.concatenate([be2, bw2])[None, :]

    gl, gh = _gather_sc(zl_pad, zh_pad, src, tgt)
    wm = _edge_mlp(gl, gh, w1c, b1, w2c, b2c)
    partials = _scatter_sc(wm, tgt)
    delta = _node_mlp(zh, partials, Wn1, bn1[None, :], Wn2, bn2[None, :])
    return delta[None]


# speedup vs baseline: 4.3396x; 2.8980x over previous
import functools

import jax
import jax.numpy as jnp
from jax import lax
from jax.experimental import pallas as pl
from jax.experimental.pallas import tpu as pltpu
from jax.experimental.pallas import tpu_sc as plsc

FD = 13
MD = 16
HD = 32
PAD = 16

N_NODES = 100000
E_EDGES = 1600000

NC = 2
NS = 16
NW = NC * NS
PER_W = E_EDGES // NW
CHUNK = 2000

RPT = N_NODES // NS
ZB = 250
SCHUNK = 1000

EBLK = 8000
NBLK = 10000

_mesh = plsc.VectorSubcoreMesh(core_axis_name="c", subcore_axis_name="s")
_sc_params = pltpu.CompilerParams(use_tc_tiling_on_sc=False)


@functools.partial(
    pl.kernel,
    mesh=_mesh,
    out_type=[
        jax.ShapeDtypeStruct((E_EDGES, PAD), jnp.float32),
        jax.ShapeDtypeStruct((E_EDGES, PAD), jnp.float32),
    ],
    scratch_types=[
        pltpu.VMEM((CHUNK,), jnp.int32),
        pltpu.VMEM((CHUNK,), jnp.int32),
        pltpu.VMEM((CHUNK, PAD), jnp.float32),
        pltpu.VMEM((CHUNK, PAD), jnp.float32),
        pltpu.SemaphoreType.DMA,
    ],
    compiler_params=_sc_params,
)
def _gather_sc(zl_hbm, zh_hbm, src_hbm, tgt_hbm, outl_hbm, outh_hbm,
               idx_s, idx_t, rows_l, rows_h, sem):
    wid = lax.axis_index("s") * NC + lax.axis_index("c")
    base0 = wid * PER_W

    @pl.loop(0, PER_W, step=CHUNK)
    def _(off):
        base = base0 + off
        pltpu.sync_copy(src_hbm.at[pl.ds(base, CHUNK)], idx_s)
        pltpu.sync_copy(tgt_hbm.at[pl.ds(base, CHUNK)], idx_t)
        cl = pltpu.async_copy(zl_hbm.at[idx_s], rows_l, sem)
        ch = pltpu.async_copy(zh_hbm.at[idx_t], rows_h, sem)
        cl.wait()
        ch.wait()
        pltpu.sync_copy(rows_l, outl_hbm.at[pl.ds(base, CHUNK)])
        pltpu.sync_copy(rows_h, outh_hbm.at[pl.ds(base, CHUNK)])


@functools.partial(
    pl.kernel,
    mesh=_mesh,
    out_type=jax.ShapeDtypeStruct((NC, N_NODES, MD), jnp.float32),
    scratch_types=[
        pltpu.VMEM_SHARED((N_NODES, MD), jnp.float32),
        pltpu.VMEM((SCHUNK,), jnp.int32),
        pltpu.VMEM((SCHUNK, MD), jnp.float32),
        pltpu.VMEM((ZB, MD), jnp.float32),
    ],
    compiler_params=_sc_params,
)
def _scatter_sc(vals_hbm, tgt_hbm, out_hbm, acc, idx_v, vals_v, zbuf):
    cid = lax.axis_index("c")
    sid = lax.axis_index("s")

    @pl.loop(0, ZB)
    def _(i):
        zbuf.at[pl.ds(i, 1), pl.ds(0, MD)][...] = jnp.zeros((1, MD), jnp.float32)

    @pl.loop(0, RPT, step=ZB)
    def _(j):
        pltpu.sync_copy(zbuf, acc.at[pl.ds(sid * RPT + j, ZB)])

    plsc.subcore_barrier()

    base0 = (cid * NS + sid) * PER_W

    @pl.loop(0, PER_W, step=SCHUNK)
    def _(off):
        base = base0 + off
        pltpu.sync_copy(tgt_hbm.at[pl.ds(base, SCHUNK)], idx_v)
        pltpu.sync_copy(vals_hbm.at[pl.ds(base, SCHUNK)], vals_v)
        pltpu.sync_copy(vals_v, acc.at[idx_v], add=True)

    plsc.subcore_barrier()
    pltpu.sync_copy(acc.at[pl.ds(sid * RPT, RPT)],
                    out_hbm.at[cid, pl.ds(sid * RPT, RPT)])


def _edge_mlp_body(gl_ref, gh_ref, w1a_ref, w2b_ref, wc_ref, wd_ref,
                   wac_ref, b1_ref, w2_ref, b2_ref, out_ref):
    gl = gl_ref[...]
    gh = gh_ref[...]
    o1 = jnp.dot(gl, w1a_ref[...], preferred_element_type=jnp.float32)
    o2 = jnp.dot(gh, w2b_ref[...], preferred_element_type=jnp.float32)
    q = o1[:, 0:6] * o2[:, 0:6]
    cr = q[:, 0:3] - q[:, 3:6]
    d = gl[:, 0:3] - gh[:, 0:3]
    dist = jnp.sum(d * d, axis=1, keepdims=True)
    ac = jnp.sqrt(jnp.sum(cr * cr, axis=1, keepdims=True))
    pre = (o1 + o2 + b1_ref[...]
           + jnp.dot(cr, wc_ref[...], preferred_element_type=jnp.float32)
           + dist * wd_ref[...] + ac * wac_ref[...])
    h = jnp.tanh(pre)
    mw = jnp.dot(h, w2_ref[...], preferred_element_type=jnp.float32) + b2_ref[...]
    out_ref[...] = mw[:, 0:MD] * jax.nn.sigmoid(mw[:, MD:MD + 1])


def _edge_mlp(gl, gh, w1a, w2b, wc, wd, wac, b1, w2, b2):
    grid = E_EDGES // EBLK
    full = lambda a: pl.BlockSpec(a.shape, lambda i: tuple(0 for _ in a.shape))
    return pl.pallas_call(
        _edge_mlp_body,
        grid=(grid,),
        in_specs=[
            pl.BlockSpec((EBLK, PAD), lambda i: (i, 0)),
            pl.BlockSpec((EBLK, PAD), lambda i: (i, 0)),
            full(w1a), full(w2b), full(wc), full(wd), full(wac),
            full(b1), full(w2), full(b2),
        ],
        out_specs=pl.BlockSpec((EBLK, MD), lambda i: (i, 0)),
        out_shape=jax.ShapeDtypeStruct((E_EDGES, MD), jnp.float32),
    )(gl, gh, w1a, w2b, wc, wd, wac, b1, w2, b2)


def _node_mlp_body(zh_ref, p_ref, wn1_ref, bn1_ref, wn2_ref, bn2_ref, out_ref):
    zh = zh_ref[...]
    p = p_ref[...]
    magg = p[0] + p[1]
    inp = jnp.concatenate([zh, magg], axis=1)
    h = jnp.tanh(
        jnp.dot(inp, wn1_ref[...], precision=lax.Precision.HIGHEST,
                preferred_element_type=jnp.float32) + bn1_ref[...]
    )
    out_ref[...] = jnp.dot(h, wn2_ref[...], precision=lax.Precision.HIGHEST,
                           preferred_element_type=jnp.float32) + bn2_ref[...]


def _node_mlp(zh, partials, wn1, bn1, wn2, bn2):
    grid = N_NODES // NBLK
    return pl.pallas_call(
        _node_mlp_body,
        grid=(grid,),
        in_specs=[
            pl.BlockSpec((NBLK, FD), lambda i: (i, 0)),
            pl.BlockSpec((NC, NBLK, MD), lambda i: (0, i, 0)),
            pl.BlockSpec(wn1.shape, lambda i: (0, 0)),
            pl.BlockSpec(bn1.shape, lambda i: (0, 0)),
            pl.BlockSpec(wn2.shape, lambda i: (0, 0)),
            pl.BlockSpec(bn2.shape, lambda i: (0, 0)),
        ],
        out_specs=pl.BlockSpec((NBLK, FD), lambda i: (i, 0)),
        out_shape=jax.ShapeDtypeStruct((N_NODES, FD), jnp.float32),
    )(zh, partials, wn1, bn1, wn2, bn2)


def kernel(z_l, z_h, edge_index_l_h, We1, be1, We2, be2,
           Ww1, bw1, Ww2, bw2, Wn1, bn1, Wn2, bn2):
    zl = z_l[0]
    zh = z_h[0]
    src = edge_index_l_h[0, 0].astype(jnp.int32)
    tgt = edge_index_l_h[0, 1].astype(jnp.int32)

    zl_pad = jnp.pad(zl, ((0, 0), (0, PAD - FD)))
    zh_pad = jnp.pad(zh, ((0, 0), (0, PAD - FD)))

    w1 = jnp.concatenate([We1, Ww1], axis=1)
    wa = jnp.zeros((PAD, 2 * HD), jnp.float32).at[0:FD].set(w1[0:FD])
    wa = wa.at[0:3].add(w1[2 * FD:2 * FD + 3])
    wb = jnp.zeros((PAD, 2 * HD), jnp.float32).at[0:FD].set(w1[FD:2 * FD])
    wb = wb.at[0:3].add(-w1[2 * FD:2 * FD + 3])
    q1 = jnp.zeros((PAD, 6), jnp.float32)
    q2 = jnp.zeros((PAD, 6), jnp.float32)
    for j, (r1, r2) in enumerate(((4, 5), (5, 3), (3, 4))):
        q1 = q1.at[r1, j].set(1.0).at[r2, j + 3].set(1.0)
        q2 = q2.at[r2, j].set(1.0).at[r1, j + 3].set(1.0)
    w1a = jnp.concatenate([q1, wa], axis=1)
    w2b = jnp.concatenate([q2, wb], axis=1)
    pad6 = jnp.zeros((1, 6), jnp.float32)
    wc = jnp.concatenate([jnp.zeros((3, 6), jnp.float32),
                          w1[2 * FD + 4:2 * FD + 7]], axis=1)
    wd = jnp.concatenate([pad6, w1[2 * FD + 3][None]], axis=1)
    wac = jnp.concatenate([pad6, w1[2 * FD + 7][None]], axis=1)
    b1 = jnp.concatenate([pad6,
                          jnp.concatenate([be1, bw1])[None]], axis=1)
    w2c = jnp.zeros((6 + 2 * HD, MD + 1), jnp.float32)
    w2c = w2c.at[6:6 + HD, 0:MD].set(We2)
    w2c = w2c.at[6 + HD:6 + 2 * HD, MD].set(Ww2[:, 0])
    b2c = jnp.concatenate([be2, bw2])[None, :]

    gl, gh = _gather_sc(zl_pad, zh_pad, src, tgt)
    wm = _edge_mlp(gl, gh, w1a, w2b, wc, wd, wac, b1, w2c, b2c)
    partials = _scatter_sc(wm, tgt)
    delta = _node_mlp(zh, partials, Wn1, bn1[None, :], Wn2, bn2[None, :])
    return delta[None]


# speedup vs baseline: 5.9982x; 1.3822x over previous
import functools

import jax
import jax.numpy as jnp
from jax import lax
from jax.experimental import pallas as pl
from jax.experimental.pallas import tpu as pltpu
from jax.experimental.pallas import tpu_sc as plsc

FD = 13
MD = 16
HD = 32
PAD = 16

N_NODES = 100000
E_EDGES = 1600000

NC = 2
NS = 16
NW = NC * NS
PER_W = E_EDGES // NW
CHUNK = 2000

RPT = N_NODES // NS
ZB = 250
SCHUNK = 1000

EBLK = 8000
NBLK = 10000

_mesh = plsc.VectorSubcoreMesh(core_axis_name="c", subcore_axis_name="s")
_sc_params = pltpu.CompilerParams(use_tc_tiling_on_sc=False)


@functools.partial(
    pl.kernel,
    mesh=_mesh,
    out_type=[
        jax.ShapeDtypeStruct((E_EDGES, PAD), jnp.float32),
        jax.ShapeDtypeStruct((E_EDGES, PAD), jnp.float32),
    ],
    scratch_types=[
        pltpu.VMEM((CHUNK,), jnp.int32),
        pltpu.VMEM((CHUNK,), jnp.int32),
        pltpu.VMEM((CHUNK, PAD), jnp.float32),
        pltpu.VMEM((CHUNK, PAD), jnp.float32),
        pltpu.SemaphoreType.DMA,
    ],
    compiler_params=_sc_params,
)
def _gather_sc(zl_hbm, zh_hbm, src_hbm, tgt_hbm, outl_hbm, outh_hbm,
               idx_s, idx_t, rows_l, rows_h, sem):
    wid = lax.axis_index("s") * NC + lax.axis_index("c")
    base0 = wid * PER_W

    @pl.loop(0, PER_W, step=CHUNK)
    def _(off):
        base = base0 + off
        pltpu.sync_copy(src_hbm.at[pl.ds(base, CHUNK)], idx_s)
        pltpu.sync_copy(tgt_hbm.at[pl.ds(base, CHUNK)], idx_t)
        cl = pltpu.async_copy(zl_hbm.at[idx_s], rows_l, sem)
        ch = pltpu.async_copy(zh_hbm.at[idx_t], rows_h, sem)
        cl.wait()
        ch.wait()
        pltpu.sync_copy(rows_l, outl_hbm.at[pl.ds(base, CHUNK)])
        pltpu.sync_copy(rows_h, outh_hbm.at[pl.ds(base, CHUNK)])


@functools.partial(
    pl.kernel,
    mesh=_mesh,
    out_type=jax.ShapeDtypeStruct((NC, N_NODES, MD), jnp.float32),
    scratch_types=[
        pltpu.VMEM_SHARED((N_NODES, MD), jnp.float32),
        pltpu.VMEM((SCHUNK,), jnp.int32),
        pltpu.VMEM((SCHUNK, MD), jnp.float32),
        pltpu.VMEM((ZB, MD), jnp.float32),
    ],
    compiler_params=_sc_params,
)
def _scatter_sc(vals_hbm, tgt_hbm, out_hbm, acc, idx_v, vals_v, zbuf):
    cid = lax.axis_index("c")
    sid = lax.axis_index("s")

    @pl.loop(0, ZB)
    def _(i):
        zbuf.at[pl.ds(i, 1), pl.ds(0, MD)][...] = jnp.zeros((1, MD), jnp.float32)

    @pl.loop(0, RPT, step=ZB)
    def _(j):
        pltpu.sync_copy(zbuf, acc.at[pl.ds(sid * RPT + j, ZB)])

    plsc.subcore_barrier()

    base0 = (cid * NS + sid) * PER_W

    @pl.loop(0, PER_W, step=SCHUNK)
    def _(off):
        base = base0 + off
        pltpu.sync_copy(tgt_hbm.at[pl.ds(base, SCHUNK)], idx_v)
        pltpu.sync_copy(vals_hbm.at[pl.ds(base, SCHUNK)], vals_v)
        pltpu.sync_copy(vals_v, acc.at[idx_v], add=True)

    plsc.subcore_barrier()
    pltpu.sync_copy(acc.at[pl.ds(sid * RPT, RPT)],
                    out_hbm.at[cid, pl.ds(sid * RPT, RPT)])


def _edge_mlp_body(gl_ref, gh_ref, w1a_ref, w2b_ref, wc_ref, wd_ref,
                   wac_ref, b1_ref, w2_ref, b2_ref, out_ref):
    glp = gl_ref[...]
    ghp = gh_ref[...]
    gl = jnp.concatenate([glp[:, PAD * j:PAD * (j + 1)] for j in range(8)], axis=0)
    gh = jnp.concatenate([ghp[:, PAD * j:PAD * (j + 1)] for j in range(8)], axis=0)
    o1 = jnp.dot(gl, w1a_ref[...], preferred_element_type=jnp.float32)
    o2 = jnp.dot(gh, w2b_ref[...], preferred_element_type=jnp.float32)
    q = o1[:, 0:6] * o2[:, 0:6]
    cr = q[:, 0:3] - q[:, 3:6]
    d = gl[:, 0:3] - gh[:, 0:3]
    dist = jnp.sum(d * d, axis=1, keepdims=True)
    ac = jnp.sqrt(jnp.sum(cr * cr, axis=1, keepdims=True))
    pre = (o1 + o2 + b1_ref[...]
           + jnp.dot(cr, wc_ref[...], preferred_element_type=jnp.float32)
           + dist * wd_ref[...] + ac * wac_ref[...])
    h = jnp.tanh(pre)
    mw = jnp.dot(h, w2_ref[...], preferred_element_type=jnp.float32) + b2_ref[...]
    wm = mw[:, 0:MD] * jax.nn.sigmoid(mw[:, MD:MD + 1])
    r = EBLK // 8
    out_ref[...] = jnp.concatenate(
        [wm[r * j:r * (j + 1), :] for j in range(8)], axis=1)


def _edge_mlp(gl, gh, w1a, w2b, wc, wd, wac, b1, w2, b2):
    grid = E_EDGES // EBLK
    full = lambda a: pl.BlockSpec(a.shape, lambda i: tuple(0 for _ in a.shape))
    return pl.pallas_call(
        _edge_mlp_body,
        grid=(grid,),
        in_specs=[
            pl.BlockSpec((EBLK // 8, 8 * PAD), lambda i: (i, 0)),
            pl.BlockSpec((EBLK // 8, 8 * PAD), lambda i: (i, 0)),
            full(w1a), full(w2b), full(wc), full(wd), full(wac),
            full(b1), full(w2), full(b2),
        ],
        out_specs=pl.BlockSpec((EBLK // 8, 8 * MD), lambda i: (i, 0)),
        out_shape=jax.ShapeDtypeStruct((E_EDGES // 8, 8 * MD), jnp.float32),
    )(gl, gh, w1a, w2b, wc, wd, wac, b1, w2, b2)


def _node_mlp_body(zh_ref, p_ref, wn1_ref, bn1_ref, wn2_ref, bn2_ref, out_ref):
    zh = zh_ref[...]
    p = p_ref[...]
    magg = p[0] + p[1]
    inp = jnp.concatenate([zh, magg], axis=1)
    h = jnp.tanh(
        jnp.dot(inp, wn1_ref[...], precision=lax.Precision.HIGHEST,
                preferred_element_type=jnp.float32) + bn1_ref[...]
    )
    out_ref[...] = jnp.dot(h, wn2_ref[...], precision=lax.Precision.HIGHEST,
                           preferred_element_type=jnp.float32) + bn2_ref[...]


def _node_mlp(zh, partials, wn1, bn1, wn2, bn2):
    grid = N_NODES // NBLK
    return pl.pallas_call(
        _node_mlp_body,
        grid=(grid,),
        in_specs=[
            pl.BlockSpec((NBLK, FD), lambda i: (i, 0)),
            pl.BlockSpec((NC, NBLK, MD), lambda i: (0, i, 0)),
            pl.BlockSpec(wn1.shape, lambda i: (0, 0)),
            pl.BlockSpec(bn1.shape, lambda i: (0, 0)),
            pl.BlockSpec(wn2.shape, lambda i: (0, 0)),
            pl.BlockSpec(bn2.shape, lambda i: (0, 0)),
        ],
        out_specs=pl.BlockSpec((NBLK, FD), lambda i: (i, 0)),
        out_shape=jax.ShapeDtypeStruct((N_NODES, FD), jnp.float32),
    )(zh, partials, wn1, bn1, wn2, bn2)


def kernel(z_l, z_h, edge_index_l_h, We1, be1, We2, be2,
           Ww1, bw1, Ww2, bw2, Wn1, bn1, Wn2, bn2):
    zl = z_l[0]
    zh = z_h[0]
    src = edge_index_l_h[0, 0].astype(jnp.int32)
    tgt = edge_index_l_h[0, 1].astype(jnp.int32)

    zl_pad = jnp.pad(zl, ((0, 0), (0, PAD - FD)))
    zh_pad = jnp.pad(zh, ((0, 0), (0, PAD - FD)))

    w1 = jnp.concatenate([We1, Ww1], axis=1)
    wa = jnp.zeros((PAD, 2 * HD), jnp.float32).at[0:FD].set(w1[0:FD])
    wa = wa.at[0:3].add(w1[2 * FD:2 * FD + 3])
    wb = jnp.zeros((PAD, 2 * HD), jnp.float32).at[0:FD].set(w1[FD:2 * FD])
    wb = wb.at[0:3].add(-w1[2 * FD:2 * FD + 3])
    q1 = jnp.zeros((PAD, 6), jnp.float32)
    q2 = jnp.zeros((PAD, 6), jnp.float32)
    for j, (r1, r2) in enumerate(((4, 5), (5, 3), (3, 4))):
        q1 = q1.at[r1, j].set(1.0).at[r2, j + 3].set(1.0)
        q2 = q2.at[r2, j].set(1.0).at[r1, j + 3].set(1.0)
    w1a = jnp.concatenate([q1, wa], axis=1)
    w2b = jnp.concatenate([q2, wb], axis=1)
    pad6 = jnp.zeros((1, 6), jnp.float32)
    wc = jnp.concatenate([jnp.zeros((3, 6), jnp.float32),
                          w1[2 * FD + 4:2 * FD + 7]], axis=1)
    wd = jnp.concatenate([pad6, w1[2 * FD + 3][None]], axis=1)
    wac = jnp.concatenate([pad6, w1[2 * FD + 7][None]], axis=1)
    b1 = jnp.concatenate([pad6,
                          jnp.concatenate([be1, bw1])[None]], axis=1)
    w2c = jnp.zeros((6 + 2 * HD, MD + 1), jnp.float32)
    w2c = w2c.at[6:6 + HD, 0:MD].set(We2)
    w2c = w2c.at[6 + HD:6 + 2 * HD, MD].set(Ww2[:, 0])
    b2c = jnp.concatenate([be2, bw2])[None, :]

    gl, gh = _gather_sc(zl_pad, zh_pad, src, tgt)
    gl = gl.reshape(E_EDGES // 8, 8 * PAD)
    gh = gh.reshape(E_EDGES // 8, 8 * PAD)
    wm = _edge_mlp(gl, gh, w1a, w2b, wc, wd, wac, b1, w2c, b2c)
    partials = _scatter_sc(wm.reshape(E_EDGES, MD), tgt)
    delta = _node_mlp(zh, partials, Wn1, bn1[None, :], Wn2, bn2[None, :])
    return delta[None]


# speedup vs baseline: 6.0635x; 1.0109x over previous
import functools

import jax
import jax.numpy as jnp
from jax import lax
from jax.experimental import pallas as pl
from jax.experimental.pallas import tpu as pltpu
from jax.experimental.pallas import tpu_sc as plsc

FD = 13
MD = 16
HD = 32
PAD = 16

N_NODES = 100000
E_EDGES = 1600000

NC = 2
NS = 16
NW = NC * NS
PER_W = E_EDGES // NW
CHUNK = 2000

RPT = N_NODES // NS
ZB = 250
SCHUNK = 1000

NQ = 9
EBLK = 8000
NBLK = 10000

_mesh = plsc.VectorSubcoreMesh(core_axis_name="c", subcore_axis_name="s")
_sc_params = pltpu.CompilerParams(use_tc_tiling_on_sc=False)


@functools.partial(
    pl.kernel,
    mesh=_mesh,
    out_type=[
        jax.ShapeDtypeStruct((E_EDGES, PAD), jnp.float32),
        jax.ShapeDtypeStruct((E_EDGES, PAD), jnp.float32),
    ],
    scratch_types=[
        pltpu.VMEM((CHUNK,), jnp.int32),
        pltpu.VMEM((CHUNK,), jnp.int32),
        pltpu.VMEM((CHUNK, PAD), jnp.float32),
        pltpu.VMEM((CHUNK, PAD), jnp.float32),
        pltpu.SemaphoreType.DMA,
    ],
    compiler_params=_sc_params,
)
def _gather_sc(zl_hbm, zh_hbm, src_hbm, tgt_hbm, outl_hbm, outh_hbm,
               idx_s, idx_t, rows_l, rows_h, sem):
    wid = lax.axis_index("s") * NC + lax.axis_index("c")
    base0 = wid * PER_W

    @pl.loop(0, PER_W, step=CHUNK)
    def _(off):
        base = base0 + off
        pltpu.sync_copy(src_hbm.at[pl.ds(base, CHUNK)], idx_s)
        pltpu.sync_copy(tgt_hbm.at[pl.ds(base, CHUNK)], idx_t)
        cl = pltpu.async_copy(zl_hbm.at[idx_s], rows_l, sem)
        ch = pltpu.async_copy(zh_hbm.at[idx_t], rows_h, sem)
        cl.wait()
        ch.wait()
        pltpu.sync_copy(rows_l, outl_hbm.at[pl.ds(base, CHUNK)])
        pltpu.sync_copy(rows_h, outh_hbm.at[pl.ds(base, CHUNK)])


@functools.partial(
    pl.kernel,
    mesh=_mesh,
    out_type=jax.ShapeDtypeStruct((NC, N_NODES, MD), jnp.float32),
    scratch_types=[
        pltpu.VMEM_SHARED((N_NODES, MD), jnp.float32),
        pltpu.VMEM((SCHUNK,), jnp.int32),
        pltpu.VMEM((SCHUNK, MD), jnp.float32),
        pltpu.VMEM((ZB, MD), jnp.float32),
    ],
    compiler_params=_sc_params,
)
def _scatter_sc(vals_hbm, tgt_hbm, out_hbm, acc, idx_v, vals_v, zbuf):
    cid = lax.axis_index("c")
    sid = lax.axis_index("s")

    @pl.loop(0, ZB)
    def _(i):
        zbuf.at[pl.ds(i, 1), pl.ds(0, MD)][...] = jnp.zeros((1, MD), jnp.float32)

    @pl.loop(0, RPT, step=ZB)
    def _(j):
        pltpu.sync_copy(zbuf, acc.at[pl.ds(sid * RPT + j, ZB)])

    plsc.subcore_barrier()

    base0 = (cid * NS + sid) * PER_W

    @pl.loop(0, PER_W, step=SCHUNK)
    def _(off):
        base = base0 + off
        pltpu.sync_copy(tgt_hbm.at[pl.ds(base, SCHUNK)], idx_v)
        pltpu.sync_copy(vals_hbm.at[pl.ds(base, SCHUNK)], vals_v)
        pltpu.sync_copy(vals_v, acc.at[idx_v], add=True)

    plsc.subcore_barrier()
    pltpu.sync_copy(acc.at[pl.ds(sid * RPT, RPT)],
                    out_hbm.at[cid, pl.ds(sid * RPT, RPT)])


def _edge_mlp_body(gl_ref, gh_ref, w1a_ref, w2b_ref, wc_ref,
                   wac_ref, b1_ref, w2_ref, b2_ref, out_ref):
    glp = gl_ref[...]
    ghp = gh_ref[...]
    gl = jnp.concatenate([glp[:, PAD * j:PAD * (j + 1)] for j in range(8)], axis=0)
    gh = jnp.concatenate([ghp[:, PAD * j:PAD * (j + 1)] for j in range(8)], axis=0)
    o1 = jnp.dot(gl, w1a_ref[...], preferred_element_type=jnp.float32)
    o2 = jnp.dot(gh, w2b_ref[...], preferred_element_type=jnp.float32)
    q = o1[:, 0:NQ] * o2[:, 0:NQ]
    cr = q[:, 0:3] - q[:, 3:6]
    f = jnp.concatenate([cr, q[:, 6:9]], axis=1)
    ac = jnp.sqrt(jnp.sum(cr * cr, axis=1, keepdims=True))
    pre = (o1 + o2 + b1_ref[...]
           + jnp.dot(f, wc_ref[...], preferred_element_type=jnp.float32)
           + ac * wac_ref[...])
    h = jnp.tanh(pre)
    mw = jnp.dot(h, w2_ref[...], preferred_element_type=jnp.float32) + b2_ref[...]
    wm = mw[:, 0:MD] * jax.nn.sigmoid(mw[:, MD:MD + 1])
    r = EBLK // 8
    out_ref[...] = jnp.concatenate(
        [wm[r * j:r * (j + 1), :] for j in range(8)], axis=1)


def _edge_mlp(gl, gh, w1a, w2b, wc, wac, b1, w2, b2):
    grid = E_EDGES // EBLK
    full = lambda a: pl.BlockSpec(a.shape, lambda i: tuple(0 for _ in a.shape))
    return pl.pallas_call(
        _edge_mlp_body,
        grid=(grid,),
        in_specs=[
            pl.BlockSpec((EBLK // 8, 8 * PAD), lambda i: (i, 0)),
            pl.BlockSpec((EBLK // 8, 8 * PAD), lambda i: (i, 0)),
            full(w1a), full(w2b), full(wc), full(wac),
            full(b1), full(w2), full(b2),
        ],
        out_specs=pl.BlockSpec((EBLK // 8, 8 * MD), lambda i: (i, 0)),
        out_shape=jax.ShapeDtypeStruct((E_EDGES // 8, 8 * MD), jnp.float32),
    )(gl, gh, w1a, w2b, wc, wac, b1, w2, b2)


def _node_mlp_body(zh_ref, p_ref, wn1_ref, bn1_ref, wn2_ref, bn2_ref, out_ref):
    zh = zh_ref[...]
    p = p_ref[...]
    magg = p[0] + p[1]
    inp = jnp.concatenate([zh, magg], axis=1)
    h = jnp.tanh(
        jnp.dot(inp, wn1_ref[...], precision=lax.Precision.HIGHEST,
                preferred_element_type=jnp.float32) + bn1_ref[...]
    )
    out_ref[...] = jnp.dot(h, wn2_ref[...], precision=lax.Precision.HIGHEST,
                           preferred_element_type=jnp.float32) + bn2_ref[...]


def _node_mlp(zh, partials, wn1, bn1, wn2, bn2):
    grid = N_NODES // NBLK
    return pl.pallas_call(
        _node_mlp_body,
        grid=(grid,),
        in_specs=[
            pl.BlockSpec((NBLK, FD), lambda i: (i, 0)),
            pl.BlockSpec((NC, NBLK, MD), lambda i: (0, i, 0)),
            pl.BlockSpec(wn1.shape, lambda i: (0, 0)),
            pl.BlockSpec(bn1.shape, lambda i: (0, 0)),
            pl.BlockSpec(wn2.shape, lambda i: (0, 0)),
            pl.BlockSpec(bn2.shape, lambda i: (0, 0)),
        ],
        out_specs=pl.BlockSpec((NBLK, FD), lambda i: (i, 0)),
        out_shape=jax.ShapeDtypeStruct((N_NODES, FD), jnp.float32),
    )(zh, partials, wn1, bn1, wn2, bn2)


def kernel(z_l, z_h, edge_index_l_h, We1, be1, We2, be2,
           Ww1, bw1, Ww2, bw2, Wn1, bn1, Wn2, bn2):
    zl = z_l[0]
    zh = z_h[0]
    src = edge_index_l_h[0, 0].astype(jnp.int32)
    tgt = edge_index_l_h[0, 1].astype(jnp.int32)

    zl_pad = jnp.concatenate(
        [zl, jnp.sum(zl[:, 0:3] ** 2, axis=1, keepdims=True),
         jnp.zeros((N_NODES, PAD - FD - 1), jnp.float32)], axis=1)
    zh_pad = jnp.concatenate(
        [zh, jnp.sum(zh[:, 0:3] ** 2, axis=1, keepdims=True),
         jnp.zeros((N_NODES, PAD - FD - 1), jnp.float32)], axis=1)

    w1 = jnp.concatenate([We1, Ww1], axis=1)
    wdrow = w1[2 * FD + 3]
    wa = jnp.zeros((PAD, 2 * HD), jnp.float32).at[0:FD].set(w1[0:FD])
    wa = wa.at[0:3].add(w1[2 * FD:2 * FD + 3])
    wa = wa.at[FD].set(wdrow)
    wb = jnp.zeros((PAD, 2 * HD), jnp.float32).at[0:FD].set(w1[FD:2 * FD])
    wb = wb.at[0:3].add(-w1[2 * FD:2 * FD + 3])
    wb = wb.at[FD].set(wdrow)
    q1 = jnp.zeros((PAD, NQ), jnp.float32)
    q2 = jnp.zeros((PAD, NQ), jnp.float32)
    for j, (r1, r2) in enumerate(((4, 5), (5, 3), (3, 4))):
        q1 = q1.at[r1, j].set(1.0).at[r2, j + 3].set(1.0)
        q2 = q2.at[r2, j].set(1.0).at[r1, j + 3].set(1.0)
    for r in range(3):
        q1 = q1.at[r, 6 + r].set(1.0)
        q2 = q2.at[r, 6 + r].set(1.0)
    w1a = jnp.concatenate([q1, wa], axis=1)
    w2b = jnp.concatenate([q2, wb], axis=1)
    padq = jnp.zeros((1, NQ), jnp.float32)
    wf = jnp.concatenate(
        [jnp.zeros((6, NQ), jnp.float32),
         jnp.concatenate([w1[2 * FD + 4:2 * FD + 7],
                          jnp.tile(-2.0 * wdrow[None], (3, 1))], axis=0)],
        axis=1)
    wac = jnp.concatenate([padq, w1[2 * FD + 7][None]], axis=1)
    b1 = jnp.concatenate([padq,
                          jnp.concatenate([be1, bw1])[None]], axis=1)
    w2c = jnp.zeros((NQ + 2 * HD, MD + 1), jnp.float32)
    w2c = w2c.at[NQ:NQ + HD, 0:MD].set(We2)
    w2c = w2c.at[NQ + HD:NQ + 2 * HD, MD].set(Ww2[:, 0])
    b2c = jnp.concatenate([be2, bw2])[None, :]

    gl, gh = _gather_sc(zl_pad, zh_pad, src, tgt)
    gl = gl.reshape(E_EDGES // 8, 8 * PAD)
    gh = gh.reshape(E_EDGES // 8, 8 * PAD)
    wm = _edge_mlp(gl, gh, w1a, w2b, wf, wac, b1, w2c, b2c)
    partials = _scatter_sc(wm.reshape(E_EDGES, MD), tgt)
    delta = _node_mlp(zh, partials, Wn1, bn1[None, :], Wn2, bn2[None, :])
    return delta[None]


# speedup vs baseline: 6.0980x; 1.0057x over previous
import functools

import jax
import jax.numpy as jnp
from jax import lax
from jax.experimental import pallas as pl
from jax.experimental.pallas import tpu as pltpu
from jax.experimental.pallas import tpu_sc as plsc

FD = 13
MD = 16
HD = 32
PAD = 16

N_NODES = 100000
E_EDGES = 1600000

NC = 2
NS = 16
NW = NC * NS
PER_W = E_EDGES // NW
CHUNK = 2000

RPT = N_NODES // NS
ZB = 250
SCHUNK = 1000

NQ = 9
EBLK = 16000
NBLK = 10000

_mesh = plsc.VectorSubcoreMesh(core_axis_name="c", subcore_axis_name="s")
_sc_params = pltpu.CompilerParams(use_tc_tiling_on_sc=False)


@functools.partial(
    pl.kernel,
    mesh=_mesh,
    out_type=[
        jax.ShapeDtypeStruct((E_EDGES, PAD), jnp.float32),
        jax.ShapeDtypeStruct((E_EDGES, PAD), jnp.float32),
    ],
    scratch_types=[
        pltpu.VMEM((CHUNK,), jnp.int32),
        pltpu.VMEM((CHUNK,), jnp.int32),
        pltpu.VMEM((CHUNK, PAD), jnp.float32),
        pltpu.VMEM((CHUNK, PAD), jnp.float32),
        pltpu.SemaphoreType.DMA,
    ],
    compiler_params=_sc_params,
)
def _gather_sc(zl_hbm, zh_hbm, src_hbm, tgt_hbm, outl_hbm, outh_hbm,
               idx_s, idx_t, rows_l, rows_h, sem):
    wid = lax.axis_index("s") * NC + lax.axis_index("c")
    base0 = wid * PER_W

    @pl.loop(0, PER_W, step=CHUNK)
    def _(off):
        base = base0 + off
        pltpu.sync_copy(src_hbm.at[pl.ds(base, CHUNK)], idx_s)
        pltpu.sync_copy(tgt_hbm.at[pl.ds(base, CHUNK)], idx_t)
        cl = pltpu.async_copy(zl_hbm.at[idx_s], rows_l, sem)
        ch = pltpu.async_copy(zh_hbm.at[idx_t], rows_h, sem)
        cl.wait()
        ch.wait()
        pltpu.sync_copy(rows_l, outl_hbm.at[pl.ds(base, CHUNK)])
        pltpu.sync_copy(rows_h, outh_hbm.at[pl.ds(base, CHUNK)])


@functools.partial(
    pl.kernel,
    mesh=_mesh,
    out_type=jax.ShapeDtypeStruct((NC, N_NODES, MD), jnp.float32),
    scratch_types=[
        pltpu.VMEM_SHARED((N_NODES, MD), jnp.float32),
        pltpu.VMEM((SCHUNK,), jnp.int32),
        pltpu.VMEM((SCHUNK, MD), jnp.float32),
        pltpu.VMEM((ZB, MD), jnp.float32),
    ],
    compiler_params=_sc_params,
)
def _scatter_sc(vals_hbm, tgt_hbm, out_hbm, acc, idx_v, vals_v, zbuf):
    cid = lax.axis_index("c")
    sid = lax.axis_index("s")

    @pl.loop(0, ZB)
    def _(i):
        zbuf.at[pl.ds(i, 1), pl.ds(0, MD)][...] = jnp.zeros((1, MD), jnp.float32)

    @pl.loop(0, RPT, step=ZB)
    def _(j):
        pltpu.sync_copy(zbuf, acc.at[pl.ds(sid * RPT + j, ZB)])

    plsc.subcore_barrier()

    base0 = (cid * NS + sid) * PER_W

    @pl.loop(0, PER_W, step=SCHUNK)
    def _(off):
        base = base0 + off
        pltpu.sync_copy(tgt_hbm.at[pl.ds(base, SCHUNK)], idx_v)
        pltpu.sync_copy(vals_hbm.at[pl.ds(base, SCHUNK)], vals_v)
        pltpu.sync_copy(vals_v, acc.at[idx_v], add=True)

    plsc.subcore_barrier()
    pltpu.sync_copy(acc.at[pl.ds(sid * RPT, RPT)],
                    out_hbm.at[cid, pl.ds(sid * RPT, RPT)])


def _edge_mlp_body(gl_ref, gh_ref, w1a_ref, w2b_ref, wc_ref,
                   wac_ref, b1_ref, w2_ref, b2_ref, out_ref):
    glp = gl_ref[...]
    ghp = gh_ref[...]
    gl = jnp.concatenate([glp[:, PAD * j:PAD * (j + 1)] for j in range(8)], axis=0)
    gh = jnp.concatenate([ghp[:, PAD * j:PAD * (j + 1)] for j in range(8)], axis=0)
    o1 = jnp.dot(gl, w1a_ref[...], preferred_element_type=jnp.float32)
    o2 = jnp.dot(gh, w2b_ref[...], preferred_element_type=jnp.float32)
    q = o1[:, 0:NQ] * o2[:, 0:NQ]
    cr = q[:, 0:3] - q[:, 3:6]
    f = jnp.concatenate([cr, q[:, 6:9]], axis=1)
    ac = jnp.sqrt(jnp.sum(cr * cr, axis=1, keepdims=True))
    pre = (o1 + o2 + b1_ref[...]
           + jnp.dot(f, wc_ref[...], preferred_element_type=jnp.float32)
           + ac * wac_ref[...])
    h = jnp.tanh(pre)
    mw = jnp.dot(h, w2_ref[...], preferred_element_type=jnp.float32) + b2_ref[...]
    wm = mw[:, 0:MD] * jax.nn.sigmoid(mw[:, MD:MD + 1])
    r = EBLK // 8
    out_ref[...] = jnp.concatenate(
        [wm[r * j:r * (j + 1), :] for j in range(8)], axis=1)


def _edge_mlp(gl, gh, w1a, w2b, wc, wac, b1, w2, b2):
    grid = E_EDGES // EBLK
    full = lambda a: pl.BlockSpec(a.shape, lambda i: tuple(0 for _ in a.shape))
    return pl.pallas_call(
        _edge_mlp_body,
        grid=(grid,),
        in_specs=[
            pl.BlockSpec((EBLK // 8, 8 * PAD), lambda i: (i, 0)),
            pl.BlockSpec((EBLK // 8, 8 * PAD), lambda i: (i, 0)),
            full(w1a), full(w2b), full(wc), full(wac),
            full(b1), full(w2), full(b2),
        ],
        out_specs=pl.BlockSpec((EBLK // 8, 8 * MD), lambda i: (i, 0)),
        out_shape=jax.ShapeDtypeStruct((E_EDGES // 8, 8 * MD), jnp.float32),
    )(gl, gh, w1a, w2b, wc, wac, b1, w2, b2)


def _node_mlp_body(zh_ref, p_ref, wn1_ref, bn1_ref, wn2_ref, bn2_ref, out_ref):
    zh = zh_ref[...]
    p = p_ref[...]
    magg = p[0] + p[1]
    inp = jnp.concatenate([zh, magg], axis=1)
    h = jnp.tanh(
        jnp.dot(inp, wn1_ref[...], precision=lax.Precision.HIGHEST,
                preferred_element_type=jnp.float32) + bn1_ref[...]
    )
    out_ref[...] = jnp.dot(h, wn2_ref[...], precision=lax.Precision.HIGHEST,
                           preferred_element_type=jnp.float32) + bn2_ref[...]


def _node_mlp(zh, partials, wn1, bn1, wn2, bn2):
    grid = N_NODES // NBLK
    return pl.pallas_call(
        _node_mlp_body,
        grid=(grid,),
        in_specs=[
            pl.BlockSpec((NBLK, FD), lambda i: (i, 0)),
            pl.BlockSpec((NC, NBLK, MD), lambda i: (0, i, 0)),
            pl.BlockSpec(wn1.shape, lambda i: (0, 0)),
            pl.BlockSpec(bn1.shape, lambda i: (0, 0)),
            pl.BlockSpec(wn2.shape, lambda i: (0, 0)),
            pl.BlockSpec(bn2.shape, lambda i: (0, 0)),
        ],
        out_specs=pl.BlockSpec((NBLK, FD), lambda i: (i, 0)),
        out_shape=jax.ShapeDtypeStruct((N_NODES, FD), jnp.float32),
    )(zh, partials, wn1, bn1, wn2, bn2)


def kernel(z_l, z_h, edge_index_l_h, We1, be1, We2, be2,
           Ww1, bw1, Ww2, bw2, Wn1, bn1, Wn2, bn2):
    zl = z_l[0]
    zh = z_h[0]
    src = edge_index_l_h[0, 0].astype(jnp.int32)
    tgt = edge_index_l_h[0, 1].astype(jnp.int32)

    zl_pad = jnp.concatenate(
        [zl, jnp.sum(zl[:, 0:3] ** 2, axis=1, keepdims=True),
         jnp.zeros((N_NODES, PAD - FD - 1), jnp.float32)], axis=1)
    zh_pad = jnp.concatenate(
        [zh, jnp.sum(zh[:, 0:3] ** 2, axis=1, keepdims=True),
         jnp.zeros((N_NODES, PAD - FD - 1), jnp.float32)], axis=1)

    w1 = jnp.concatenate([We1, Ww1], axis=1)
    wdrow = w1[2 * FD + 3]
    wa = jnp.zeros((PAD, 2 * HD), jnp.float32).at[0:FD].set(w1[0:FD])
    wa = wa.at[0:3].add(w1[2 * FD:2 * FD + 3])
    wa = wa.at[FD].set(wdrow)
    wb = jnp.zeros((PAD, 2 * HD), jnp.float32).at[0:FD].set(w1[FD:2 * FD])
    wb = wb.at[0:3].add(-w1[2 * FD:2 * FD + 3])
    wb = wb.at[FD].set(wdrow)
    q1 = jnp.zeros((PAD, NQ), jnp.float32)
    q2 = jnp.zeros((PAD, NQ), jnp.float32)
    for j, (r1, r2) in enumerate(((4, 5), (5, 3), (3, 4))):
        q1 = q1.at[r1, j].set(1.0).at[r2, j + 3].set(1.0)
        q2 = q2.at[r2, j].set(1.0).at[r1, j + 3].set(1.0)
    for r in range(3):
        q1 = q1.at[r, 6 + r].set(1.0)
        q2 = q2.at[r, 6 + r].set(1.0)
    w1a = jnp.concatenate([q1, wa], axis=1)
    w2b = jnp.concatenate([q2, wb], axis=1)
    padq = jnp.zeros((1, NQ), jnp.float32)
    wf = jnp.concatenate(
        [jnp.zeros((6, NQ), jnp.float32),
         jnp.concatenate([w1[2 * FD + 4:2 * FD + 7],
                          jnp.tile(-2.0 * wdrow[None], (3, 1))], axis=0)],
        axis=1)
    wac = jnp.concatenate([padq, w1[2 * FD + 7][None]], axis=1)
    b1 = jnp.concatenate([padq,
                          jnp.concatenate([be1, bw1])[None]], axis=1)
    w2c = jnp.zeros((NQ + 2 * HD, MD + 1), jnp.float32)
    w2c = w2c.at[NQ:NQ + HD, 0:MD].set(We2)
    w2c = w2c.at[NQ + HD:NQ + 2 * HD, MD].set(Ww2[:, 0])
    b2c = jnp.concatenate([be2, bw2])[None, :]

    gl, gh = _gather_sc(zl_pad, zh_pad, src, tgt)
    gl = gl.reshape(E_EDGES // 8, 8 * PAD)
    gh = gh.reshape(E_EDGES // 8, 8 * PAD)
    wm = _edge_mlp(gl, gh, w1a, w2b, wf, wac, b1, w2c, b2c)
    partials = _scatter_sc(wm.reshape(E_EDGES, MD), tgt)
    delta = _node_mlp(zh, partials, Wn1, bn1[None, :], Wn2, bn2[None, :])
    return delta[None]


# speedup vs baseline: 6.3468x; 1.0408x over previous
import functools

import jax
import jax.numpy as jnp
from jax import lax
from jax.experimental import pallas as pl
from jax.experimental.pallas import tpu as pltpu
from jax.experimental.pallas import tpu_sc as plsc

FD = 13
MD = 16
HD = 32
PAD = 16

N_NODES = 100000
E_EDGES = 1600000

NC = 2
NS = 16
NW = NC * NS
PER_W = E_EDGES // NW
CHUNK = 2000

RPT = N_NODES // NS
ZB = 250
SCHUNK = 1000

NQ = 9
EBLK = 16000
NBLK = 10000

_mesh = plsc.VectorSubcoreMesh(core_axis_name="c", subcore_axis_name="s")
_sc_params = pltpu.CompilerParams(use_tc_tiling_on_sc=False)


@functools.partial(
    pl.kernel,
    mesh=_mesh,
    out_type=[
        jax.ShapeDtypeStruct((E_EDGES, PAD), jnp.float32),
        jax.ShapeDtypeStruct((E_EDGES, PAD), jnp.float32),
    ],
    scratch_types=[
        pltpu.VMEM((CHUNK,), jnp.int32),
        pltpu.VMEM((CHUNK,), jnp.int32),
        pltpu.VMEM((CHUNK, PAD), jnp.float32),
        pltpu.VMEM((CHUNK, PAD), jnp.float32),
        pltpu.SemaphoreType.DMA,
    ],
    compiler_params=_sc_params,
)
def _gather_sc(zl_hbm, zh_hbm, src_hbm, tgt_hbm, outl_hbm, outh_hbm,
               idx_s, idx_t, rows_l, rows_h, sem):
    wid = lax.axis_index("s") * NC + lax.axis_index("c")
    base0 = wid * PER_W

    @pl.loop(0, PER_W, step=CHUNK)
    def _(off):
        base = base0 + off
        pltpu.sync_copy(src_hbm.at[pl.ds(base, CHUNK)], idx_s)
        pltpu.sync_copy(tgt_hbm.at[pl.ds(base, CHUNK)], idx_t)
        cl = pltpu.async_copy(zl_hbm.at[idx_s], rows_l, sem)
        ch = pltpu.async_copy(zh_hbm.at[idx_t], rows_h, sem)
        cl.wait()
        ch.wait()
        pltpu.sync_copy(rows_l, outl_hbm.at[pl.ds(base, CHUNK)])
        pltpu.sync_copy(rows_h, outh_hbm.at[pl.ds(base, CHUNK)])


@functools.partial(
    pl.kernel,
    mesh=_mesh,
    out_type=jax.ShapeDtypeStruct((NC, N_NODES, MD), jnp.float32),
    scratch_types=[
        pltpu.VMEM_SHARED((N_NODES, MD), jnp.float32),
        pltpu.VMEM((SCHUNK,), jnp.int32),
        pltpu.VMEM((SCHUNK, MD), jnp.float32),
        pltpu.VMEM((ZB, MD), jnp.float32),
    ],
    compiler_params=_sc_params,
)
def _scatter_sc(vals_hbm, tgt_hbm, out_hbm, acc, idx_v, vals_v, zbuf):
    cid = lax.axis_index("c")
    sid = lax.axis_index("s")

    @pl.loop(0, ZB)
    def _(i):
        zbuf.at[pl.ds(i, 1), pl.ds(0, MD)][...] = jnp.zeros((1, MD), jnp.float32)

    @pl.loop(0, RPT, step=ZB)
    def _(j):
        pltpu.sync_copy(zbuf, acc.at[pl.ds(sid * RPT + j, ZB)])

    plsc.subcore_barrier()

    base0 = (cid * NS + sid) * PER_W

    @pl.loop(0, PER_W, step=SCHUNK)
    def _(off):
        base = base0 + off
        pltpu.sync_copy(tgt_hbm.at[pl.ds(base, SCHUNK)], idx_v)
        pltpu.sync_copy(vals_hbm.at[pl.ds(base, SCHUNK)], vals_v)
        pltpu.sync_copy(vals_v, acc.at[idx_v], add=True)

    plsc.subcore_barrier()
    pltpu.sync_copy(acc.at[pl.ds(sid * RPT, RPT)],
                    out_hbm.at[cid, pl.ds(sid * RPT, RPT)])


def _edge_mlp_body(gl_ref, gh_ref, w1a_ref, w2b_ref, wc_ref,
                   wac_ref, b1_ref, w2_ref, b2_ref, out_ref):
    glp = gl_ref[...]
    ghp = gh_ref[...]
    gl = jnp.concatenate([glp[:, PAD * j:PAD * (j + 1)] for j in range(8)], axis=0)
    gh = jnp.concatenate([ghp[:, PAD * j:PAD * (j + 1)] for j in range(8)], axis=0)
    o1 = jnp.dot(gl, w1a_ref[...], preferred_element_type=jnp.float32)
    o2 = jnp.dot(gh, w2b_ref[...], preferred_element_type=jnp.float32)
    q = o1[:, 0:NQ] * o2[:, 0:NQ]
    cr = q[:, 0:3] - q[:, 3:6]
    f = jnp.concatenate([cr, q[:, 6:9]], axis=1)
    ac = jnp.sqrt(jnp.sum(cr * cr, axis=1, keepdims=True))
    pre = (o1 + o2 + b1_ref[...]
           + jnp.dot(f, wc_ref[...], preferred_element_type=jnp.float32)
           + ac * wac_ref[...])
    h = jnp.tanh(pre)
    mw = jnp.dot(h, w2_ref[...], preferred_element_type=jnp.float32) + b2_ref[...]
    wm = mw[:, 0:MD] * jax.nn.sigmoid(mw[:, MD:MD + 1])
    r = EBLK // 8
    out_ref[...] = jnp.concatenate(
        [wm[r * j:r * (j + 1), :] for j in range(8)], axis=1)


def _edge_mlp(gl, gh, w1a, w2b, wc, wac, b1, w2, b2):
    grid = E_EDGES // EBLK
    full = lambda a: pl.BlockSpec(a.shape, lambda i: tuple(0 for _ in a.shape))
    return pl.pallas_call(
        _edge_mlp_body,
        grid=(grid,),
        in_specs=[
            pl.BlockSpec((EBLK // 8, 8 * PAD), lambda i: (i, 0)),
            pl.BlockSpec((EBLK // 8, 8 * PAD), lambda i: (i, 0)),
            full(w1a), full(w2b), full(wc), full(wac),
            full(b1), full(w2), full(b2),
        ],
        out_specs=pl.BlockSpec((EBLK // 8, 8 * MD), lambda i: (i, 0)),
        out_shape=jax.ShapeDtypeStruct((E_EDGES // 8, 8 * MD), jnp.float32),
    )(gl, gh, w1a, w2b, wc, wac, b1, w2, b2)


def _node_mlp_body(zh_ref, p_ref, wn1_ref, bn1_ref, wn2_ref, bn2_ref, out_ref):
    zh = zh_ref[...]
    p = p_ref[...]
    magg = p[0] + p[1]
    inp = jnp.concatenate([zh, magg], axis=1)
    h = jnp.tanh(
        jnp.dot(inp, wn1_ref[...], preferred_element_type=jnp.float32)
        + bn1_ref[...]
    )
    out_ref[0] = jnp.dot(h, wn2_ref[...],
                         preferred_element_type=jnp.float32) + bn2_ref[...]


def _node_mlp(zh, partials, wn1, bn1, wn2, bn2):
    grid = N_NODES // NBLK
    return pl.pallas_call(
        _node_mlp_body,
        grid=(grid,),
        in_specs=[
            pl.BlockSpec((NBLK, FD), lambda i: (i, 0)),
            pl.BlockSpec((NC, NBLK, MD), lambda i: (0, i, 0)),
            pl.BlockSpec(wn1.shape, lambda i: (0, 0)),
            pl.BlockSpec(bn1.shape, lambda i: (0, 0)),
            pl.BlockSpec(wn2.shape, lambda i: (0, 0)),
            pl.BlockSpec(bn2.shape, lambda i: (0, 0)),
        ],
        out_specs=pl.BlockSpec((1, NBLK, FD), lambda i: (0, i, 0)),
        out_shape=jax.ShapeDtypeStruct((1, N_NODES, FD), jnp.float32),
    )(zh, partials, wn1, bn1, wn2, bn2)


def kernel(z_l, z_h, edge_index_l_h, We1, be1, We2, be2,
           Ww1, bw1, Ww2, bw2, Wn1, bn1, Wn2, bn2):
    zl = z_l[0]
    zh = z_h[0]
    src = edge_index_l_h[0, 0].astype(jnp.int32)
    tgt = edge_index_l_h[0, 1].astype(jnp.int32)

    zl_pad = jnp.concatenate(
        [zl, jnp.sum(zl[:, 0:3] ** 2, axis=1, keepdims=True),
         jnp.zeros((N_NODES, PAD - FD - 1), jnp.float32)], axis=1)
    zh_pad = jnp.concatenate(
        [zh, jnp.sum(zh[:, 0:3] ** 2, axis=1, keepdims=True),
         jnp.zeros((N_NODES, PAD - FD - 1), jnp.float32)], axis=1)

    w1 = jnp.concatenate([We1, Ww1], axis=1)
    wdrow = w1[2 * FD + 3]
    wa = jnp.zeros((PAD, 2 * HD), jnp.float32).at[0:FD].set(w1[0:FD])
    wa = wa.at[0:3].add(w1[2 * FD:2 * FD + 3])
    wa = wa.at[FD].set(wdrow)
    wb = jnp.zeros((PAD, 2 * HD), jnp.float32).at[0:FD].set(w1[FD:2 * FD])
    wb = wb.at[0:3].add(-w1[2 * FD:2 * FD + 3])
    wb = wb.at[FD].set(wdrow)
    q1 = jnp.zeros((PAD, NQ), jnp.float32)
    q2 = jnp.zeros((PAD, NQ), jnp.float32)
    for j, (r1, r2) in enumerate(((4, 5), (5, 3), (3, 4))):
        q1 = q1.at[r1, j].set(1.0).at[r2, j + 3].set(1.0)
        q2 = q2.at[r2, j].set(1.0).at[r1, j + 3].set(1.0)
    for r in range(3):
        q1 = q1.at[r, 6 + r].set(1.0)
        q2 = q2.at[r, 6 + r].set(1.0)
    w1a = jnp.concatenate([q1, wa], axis=1)
    w2b = jnp.concatenate([q2, wb], axis=1)
    padq = jnp.zeros((1, NQ), jnp.float32)
    wf = jnp.concatenate(
        [jnp.zeros((6, NQ), jnp.float32),
         jnp.concatenate([w1[2 * FD + 4:2 * FD + 7],
                          jnp.tile(-2.0 * wdrow[None], (3, 1))], axis=0)],
        axis=1)
    wac = jnp.concatenate([padq, w1[2 * FD + 7][None]], axis=1)
    b1 = jnp.concatenate([padq,
                          jnp.concatenate([be1, bw1])[None]], axis=1)
    w2c = jnp.zeros((NQ + 2 * HD, MD + 1), jnp.float32)
    w2c = w2c.at[NQ:NQ + HD, 0:MD].set(We2)
    w2c = w2c.at[NQ + HD:NQ + 2 * HD, MD].set(Ww2[:, 0])
    b2c = jnp.concatenate([be2, bw2])[None, :]

    gl, gh = _gather_sc(zl_pad, zh_pad, src, tgt)
    gl = gl.reshape(E_EDGES // 8, 8 * PAD)
    gh = gh.reshape(E_EDGES // 8, 8 * PAD)
    wm = _edge_mlp(gl, gh, w1a, w2b, wf, wac, b1, w2c, b2c)
    partials = _scatter_sc(wm.reshape(E_EDGES, MD), tgt)
    return _node_mlp(zh, partials, Wn1, bn1[None, :], Wn2, bn2[None, :])


# speedup vs baseline: 6.7140x; 1.0579x over previous
import functools

import jax
import jax.numpy as jnp
from jax import lax
from jax.experimental import pallas as pl
from jax.experimental.pallas import tpu as pltpu
from jax.experimental.pallas import tpu_sc as plsc

FD = 13
MD = 16
HD = 32
PAD = 16

N_NODES = 100000
E_EDGES = 1600000

NC = 2
NS = 16
NW = NC * NS
PER_W = E_EDGES // NW
CHUNK = 2000

RPT = N_NODES // NS
ZB = 250
SCHUNK = 1000

NQ = 9
EBLK = 16000
NBLK = 10000

_mesh = plsc.VectorSubcoreMesh(core_axis_name="c", subcore_axis_name="s")
_sc_params = pltpu.CompilerParams(use_tc_tiling_on_sc=False)


@functools.partial(
    pl.kernel,
    mesh=_mesh,
    out_type=[
        jax.ShapeDtypeStruct((E_EDGES, PAD), jnp.float32),
        jax.ShapeDtypeStruct((E_EDGES, PAD), jnp.float32),
    ],
    scratch_types=[
        pltpu.VMEM((CHUNK,), jnp.int32),
        pltpu.VMEM((CHUNK,), jnp.int32),
        pltpu.VMEM((CHUNK, PAD), jnp.float32),
        pltpu.VMEM((CHUNK, PAD), jnp.float32),
        pltpu.SemaphoreType.DMA,
    ],
    compiler_params=_sc_params,
)
def _gather_sc(zl_hbm, zh_hbm, src_hbm, tgt_hbm, outl_hbm, outh_hbm,
               idx_s, idx_t, rows_l, rows_h, sem):
    wid = lax.axis_index("s") * NC + lax.axis_index("c")
    base0 = wid * PER_W

    @pl.loop(0, PER_W, step=CHUNK)
    def _(off):
        base = base0 + off
        pltpu.sync_copy(src_hbm.at[pl.ds(base, CHUNK)], idx_s)
        pltpu.sync_copy(tgt_hbm.at[pl.ds(base, CHUNK)], idx_t)
        cl = pltpu.async_copy(zl_hbm.at[idx_s], rows_l, sem)
        ch = pltpu.async_copy(zh_hbm.at[idx_t], rows_h, sem)
        cl.wait()
        ch.wait()
        pltpu.sync_copy(rows_l, outl_hbm.at[pl.ds(base, CHUNK)])
        pltpu.sync_copy(rows_h, outh_hbm.at[pl.ds(base, CHUNK)])


@functools.partial(
    pl.kernel,
    mesh=_mesh,
    out_type=jax.ShapeDtypeStruct((NC, N_NODES, MD), jnp.float32),
    scratch_types=[
        pltpu.VMEM_SHARED((N_NODES, MD), jnp.float32),
        pltpu.VMEM((SCHUNK,), jnp.int32),
        pltpu.VMEM((SCHUNK, MD), jnp.float32),
        pltpu.VMEM((ZB, MD), jnp.float32),
    ],
    compiler_params=_sc_params,
)
def _scatter_sc(vals_hbm, tgt_hbm, out_hbm, acc, idx_v, vals_v, zbuf):
    cid = lax.axis_index("c")
    sid = lax.axis_index("s")

    @pl.loop(0, ZB)
    def _(i):
        zbuf.at[pl.ds(i, 1), pl.ds(0, MD)][...] = jnp.zeros((1, MD), jnp.float32)

    @pl.loop(0, RPT, step=ZB)
    def _(j):
        pltpu.sync_copy(zbuf, acc.at[pl.ds(sid * RPT + j, ZB)])

    plsc.subcore_barrier()

    base0 = (cid * NS + sid) * PER_W

    @pl.loop(0, PER_W, step=SCHUNK)
    def _(off):
        base = base0 + off
        pltpu.sync_copy(tgt_hbm.at[pl.ds(base, SCHUNK)], idx_v)
        pltpu.sync_copy(vals_hbm.at[pl.ds(base, SCHUNK)], vals_v)
        pltpu.sync_copy(vals_v, acc.at[idx_v], add=True)

    plsc.subcore_barrier()
    pltpu.sync_copy(acc.at[pl.ds(sid * RPT, RPT)],
                    out_hbm.at[cid, pl.ds(sid * RPT, RPT)])


def _edge_mlp_body(gl_ref, gh_ref, w1a_ref, w2b_ref, wc_ref,
                   wac_ref, b1_ref, w2_ref, b2_ref, out_ref):
    glp = gl_ref[...]
    ghp = gh_ref[...]
    gl = jnp.concatenate([glp[:, PAD * j:PAD * (j + 1)] for j in range(8)], axis=0)
    gh = jnp.concatenate([ghp[:, PAD * j:PAD * (j + 1)] for j in range(8)], axis=0)
    o1 = jnp.dot(gl.astype(jnp.bfloat16), w1a_ref[...].astype(jnp.bfloat16),
                 preferred_element_type=jnp.float32)
    o2 = jnp.dot(gh.astype(jnp.bfloat16), w2b_ref[...].astype(jnp.bfloat16),
                 preferred_element_type=jnp.float32)
    q = o1[:, 0:NQ] * o2[:, 0:NQ]
    cr = q[:, 0:3] - q[:, 3:6]
    f = jnp.concatenate([cr, q[:, 6:9]], axis=1)
    ac = jnp.sqrt(jnp.sum(cr * cr, axis=1, keepdims=True))
    pre = (o1 + o2 + b1_ref[...]
           + jnp.dot(f, wc_ref[...], preferred_element_type=jnp.float32)
           + ac * wac_ref[...])
    h = jnp.tanh(pre)
    mw = jnp.dot(h, w2_ref[...], preferred_element_type=jnp.float32) + b2_ref[...]
    wm = mw[:, 0:MD] * jax.nn.sigmoid(mw[:, MD:MD + 1])
    r = EBLK // 8
    out_ref[...] = jnp.concatenate(
        [wm[r * j:r * (j + 1), :] for j in range(8)], axis=1)


def _edge_mlp(gl, gh, w1a, w2b, wc, wac, b1, w2, b2):
    grid = E_EDGES // EBLK
    full = lambda a: pl.BlockSpec(a.shape, lambda i: tuple(0 for _ in a.shape))
    return pl.pallas_call(
        _edge_mlp_body,
        grid=(grid,),
        in_specs=[
            pl.BlockSpec((EBLK // 8, 8 * PAD), lambda i: (i, 0)),
            pl.BlockSpec((EBLK // 8, 8 * PAD), lambda i: (i, 0)),
            full(w1a), full(w2b), full(wc), full(wac),
            full(b1), full(w2), full(b2),
        ],
        out_specs=pl.BlockSpec((EBLK // 8, 8 * MD), lambda i: (i, 0)),
        out_shape=jax.ShapeDtypeStruct((E_EDGES // 8, 8 * MD), jnp.float32),
    )(gl, gh, w1a, w2b, wc, wac, b1, w2, b2)


def _node_mlp_body(zh_ref, p_ref, wn1_ref, bn1_ref, wn2_ref, bn2_ref, out_ref):
    zh = zh_ref[...]
    p = p_ref[...]
    magg = p[0] + p[1]
    inp = jnp.concatenate([zh, magg], axis=1)
    h = jnp.tanh(
        jnp.dot(inp, wn1_ref[...], preferred_element_type=jnp.float32)
        + bn1_ref[...]
    )
    out_ref[0] = jnp.dot(h, wn2_ref[...],
                         preferred_element_type=jnp.float32) + bn2_ref[...]


def _node_mlp(zh, partials, wn1, bn1, wn2, bn2):
    grid = N_NODES // NBLK
    return pl.pallas_call(
        _node_mlp_body,
        grid=(grid,),
        in_specs=[
            pl.BlockSpec((NBLK, FD), lambda i: (i, 0)),
            pl.BlockSpec((NC, NBLK, MD), lambda i: (0, i, 0)),
            pl.BlockSpec(wn1.shape, lambda i: (0, 0)),
            pl.BlockSpec(bn1.shape, lambda i: (0, 0)),
            pl.BlockSpec(wn2.shape, lambda i: (0, 0)),
            pl.BlockSpec(bn2.shape, lambda i: (0, 0)),
        ],
        out_specs=pl.BlockSpec((1, NBLK, FD), lambda i: (0, i, 0)),
        out_shape=jax.ShapeDtypeStruct((1, N_NODES, FD), jnp.float32),
    )(zh, partials, wn1, bn1, wn2, bn2)


def kernel(z_l, z_h, edge_index_l_h, We1, be1, We2, be2,
           Ww1, bw1, Ww2, bw2, Wn1, bn1, Wn2, bn2):
    zl = z_l[0]
    zh = z_h[0]
    src = edge_index_l_h[0, 0].astype(jnp.int32)
    tgt = edge_index_l_h[0, 1].astype(jnp.int32)

    zl_pad = jnp.concatenate(
        [zl, jnp.sum(zl[:, 0:3] ** 2, axis=1, keepdims=True),
         jnp.zeros((N_NODES, PAD - FD - 1), jnp.float32)], axis=1)
    zh_pad = jnp.concatenate(
        [zh, jnp.sum(zh[:, 0:3] ** 2, axis=1, keepdims=True),
         jnp.zeros((N_NODES, PAD - FD - 1), jnp.float32)], axis=1)

    w1 = jnp.concatenate([We1, Ww1], axis=1)
    wdrow = w1[2 * FD + 3]
    wa = jnp.zeros((PAD, 2 * HD), jnp.float32).at[0:FD].set(w1[0:FD])
    wa = wa.at[0:3].add(w1[2 * FD:2 * FD + 3])
    wa = wa.at[FD].set(wdrow)
    wb = jnp.zeros((PAD, 2 * HD), jnp.float32).at[0:FD].set(w1[FD:2 * FD])
    wb = wb.at[0:3].add(-w1[2 * FD:2 * FD + 3])
    wb = wb.at[FD].set(wdrow)
    q1 = jnp.zeros((PAD, NQ), jnp.float32)
    q2 = jnp.zeros((PAD, NQ), jnp.float32)
    for j, (r1, r2) in enumerate(((4, 5), (5, 3), (3, 4))):
        q1 = q1.at[r1, j].set(1.0).at[r2, j + 3].set(1.0)
        q2 = q2.at[r2, j].set(1.0).at[r1, j + 3].set(1.0)
    for r in range(3):
        q1 = q1.at[r, 6 + r].set(1.0)
        q2 = q2.at[r, 6 + r].set(1.0)
    w1a = jnp.concatenate([q1, wa], axis=1)
    w2b = jnp.concatenate([q2, wb], axis=1)
    padq = jnp.zeros((1, NQ), jnp.float32)
    wf = jnp.concatenate(
        [jnp.zeros((6, NQ), jnp.float32),
         jnp.concatenate([w1[2 * FD + 4:2 * FD + 7],
                          jnp.tile(-2.0 * wdrow[None], (3, 1))], axis=0)],
        axis=1)
    wac = jnp.concatenate([padq, w1[2 * FD + 7][None]], axis=1)
    b1 = jnp.concatenate([padq,
                          jnp.concatenate([be1, bw1])[None]], axis=1)
    w2c = jnp.zeros((NQ + 2 * HD, MD + 1), jnp.float32)
    w2c = w2c.at[NQ:NQ + HD, 0:MD].set(We2)
    w2c = w2c.at[NQ + HD:NQ + 2 * HD, MD].set(Ww2[:, 0])
    b2c = jnp.concatenate([be2, bw2])[None, :]

    gl, gh = _gather_sc(zl_pad, zh_pad, src, tgt)
    gl = gl.reshape(E_EDGES // 8, 8 * PAD)
    gh = gh.reshape(E_EDGES // 8, 8 * PAD)
    wm = _edge_mlp(gl, gh, w1a, w2b, wf, wac, b1, w2c, b2c)
    partials = _scatter_sc(wm.reshape(E_EDGES, MD), tgt)
    return _node_mlp(zh, partials, Wn1, bn1[None, :], Wn2, bn2[None, :])
